# baseline (speedup 1.0000x reference)
import functools

import jax
import jax.numpy as jnp
import numpy as np
from jax import lax
from jax.experimental import pallas as pl

N_ROWS = 16384
N_CLASSES_K = 1000
COLD = np.float32(0.1 / 1000.0)
HOT = np.float32(np.float32(1.0 - 0.1) + COLD)

CB = 8
NB = N_CLASSES_K // CB


def _body(tgt_ref, out_ref):
    j = pl.program_id(0)
    cls = lax.broadcasted_iota(jnp.int32, (CB, N_ROWS), 0) + j * CB
    tgt = tgt_ref[...]
    out_ref[...] = jnp.where(tgt == cls, HOT, COLD)


def kernel(target):
    tgt2 = target.astype(jnp.int32).reshape(1, N_ROWS)
    out_t = pl.pallas_call(
        _body,
        grid=(NB,),
        in_specs=[pl.BlockSpec((1, N_ROWS), lambda j: (0, 0))],
        out_specs=pl.BlockSpec((CB, N_ROWS), lambda j: (j, 0)),
        out_shape=jax.ShapeDtypeStruct((N_CLASSES_K, N_ROWS), jnp.float32),
    )(tgt2)
    return out_t.T


# speedup vs baseline: 2.3655x; 2.3655x over previous
import functools

import jax
import jax.numpy as jnp
import numpy as np
from jax import lax
from jax.experimental import pallas as pl

N_ROWS = 16384
N_CLASSES_K = 1000
COLD = np.float32(0.1 / 1000.0)
HOT = np.float32(np.float32(1.0 - 0.1) + COLD)

CB = 200
NB = N_CLASSES_K // CB


def _body(tgt_ref, out_ref):
    j = pl.program_id(0)
    cls = lax.broadcasted_iota(jnp.int32, (CB, N_ROWS), 0) + j * CB
    tgt = tgt_ref[...]
    out_ref[...] = jnp.where(tgt == cls, HOT, COLD)


def kernel(target):
    tgt2 = target.astype(jnp.int32).reshape(1, N_ROWS)
    out_t = pl.pallas_call(
        _body,
        grid=(NB,),
        in_specs=[pl.BlockSpec((1, N_ROWS), lambda j: (0, 0))],
        out_specs=pl.BlockSpec((CB, N_ROWS), lambda j: (j, 0)),
        out_shape=jax.ShapeDtypeStruct((N_CLASSES_K, N_ROWS), jnp.float32),
    )(tgt2)
    return out_t.T


# speedup vs baseline: 2.4907x; 1.0529x over previous
import functools

import jax
import jax.numpy as jnp
import numpy as np
from jax import lax
from jax.experimental import pallas as pl

N_ROWS = 16384
N_CLASSES_K = 1000
COLD = np.float32(0.1 / 1000.0)
HOT = np.float32(np.float32(1.0 - 0.1) + COLD)

CB = 128
NB = -(-N_CLASSES_K // CB)


def _body(tgt_ref, out_ref):
    j = pl.program_id(0)
    cls = lax.broadcasted_iota(jnp.int32, (CB, N_ROWS), 0) + j * CB
    tgt = tgt_ref[...]
    out_ref[...] = jnp.where(tgt == cls, HOT, COLD)


def kernel(target):
    tgt2 = target.astype(jnp.int32).reshape(1, N_ROWS)
    out_t = pl.pallas_call(
        _body,
        grid=(NB,),
        in_specs=[pl.BlockSpec((1, N_ROWS), lambda j: (0, 0))],
        out_specs=pl.BlockSpec((CB, N_ROWS), lambda j: (j, 0)),
        out_shape=jax.ShapeDtypeStruct((N_CLASSES_K, N_ROWS), jnp.float32),
    )(tgt2)
    return out_t.T


# speedup vs baseline: 2.5972x; 1.0428x over previous
import functools

import jax
import jax.numpy as jnp
import numpy as np
from jax import lax
from jax.experimental import pallas as pl

N_ROWS = 16384
N_CLASSES_K = 1000
COLD = np.float32(0.1 / 1000.0)
HOT = np.float32(np.float32(1.0 - 0.1) + COLD)

CB = 64
NB = -(-N_CLASSES_K // CB)


def _body(tgt_ref, out_ref):
    j = pl.program_id(0)
    cls = lax.broadcasted_iota(jnp.int32, (CB, N_ROWS), 0) + j * CB
    tgt = tgt_ref[...]
    out_ref[...] = jnp.where(tgt == cls, HOT, COLD)


def kernel(target):
    tgt2 = target.astype(jnp.int32).reshape(1, N_ROWS)
    out_t = pl.pallas_call(
        _body,
        grid=(NB,),
        in_specs=[pl.BlockSpec((1, N_ROWS), lambda j: (0, 0))],
        out_specs=pl.BlockSpec((CB, N_ROWS), lambda j: (j, 0)),
        out_shape=jax.ShapeDtypeStruct((N_CLASSES_K, N_ROWS), jnp.float32),
    )(tgt2)
    return out_t.T
